# baseline (speedup 1.0000x reference)
import functools

import jax
import jax.numpy as jnp
from jax import lax
from jax.experimental import pallas as pl
from jax.experimental.pallas import tpu as pltpu, tpu_sc as plsc

_INFO = plsc.get_sparse_core_info()
_NC, _NS, _L = _INFO.num_cores, _INFO.num_subcores, _INFO.num_lanes
_NW = _NC * _NS

_B, _S = 16, 2048
_N = _B * _S
_CHUNK = _N // _NW


def _make_sc_call():
    mesh = plsc.VectorSubcoreMesh(core_axis_name="c", subcore_axis_name="s")

    @functools.partial(
        pl.kernel,
        mesh=mesh,
        out_type=jax.ShapeDtypeStruct((_N,), jnp.float32),
        scratch_types=[
            pltpu.VMEM((_CHUNK,), jnp.float32),
            pltpu.VMEM((_CHUNK,), jnp.int32),
            pltpu.VMEM((_L,), jnp.float32),
            pltpu.VMEM((_CHUNK,), jnp.float32),
        ],
    )
    def sc_kernel(scores_hbm, mask_hbm, bias_hbm, out_hbm,
                  scores_v, mask_v, bias_v, out_v):
        wid = lax.axis_index("s") * _NC + lax.axis_index("c")
        base = wid * _CHUNK
        pltpu.sync_copy(scores_hbm.at[pl.ds(base, _CHUNK)], scores_v)
        pltpu.sync_copy(mask_hbm.at[pl.ds(base, _CHUNK)], mask_v)
        pltpu.sync_copy(bias_hbm, bias_v)
        bias_vec = bias_v[...]
        for i in range(_CHUNK // _L):
            sl = pl.ds(i * _L, _L)
            s = scores_v[sl]
            m = mask_v[sl].astype(jnp.float32)
            out_v[sl] = (s + bias_vec) * m
        pltpu.sync_copy(out_v, out_hbm.at[pl.ds(base, _CHUNK)])

    return sc_kernel


_SC_CALL = _make_sc_call()


@jax.jit
def kernel(sent_group_scores, sel_sent_emb, sel_sent_masks, group_embs,
           candi_sent_masks, bias):
    scores = sent_group_scores.reshape(_N)
    mask = candi_sent_masks.reshape(_N)
    bias_vec = jnp.broadcast_to(jnp.reshape(bias, (1,)), (_L,))
    out = _SC_CALL(scores, mask, bias_vec)
    return out.reshape(_B, _S)


# speedup vs baseline: 1.0365x; 1.0365x over previous
import functools

import jax
import jax.numpy as jnp
from jax import lax
from jax.experimental import pallas as pl
from jax.experimental.pallas import tpu as pltpu, tpu_sc as plsc

_INFO = plsc.get_sparse_core_info()
_NC, _NS, _L = _INFO.num_cores, _INFO.num_subcores, _INFO.num_lanes
_NW = _NC * _NS

_B, _S = 16, 2048
_N = _B * _S
_CHUNK = _N // _NW


def _make_sc_call():
    mesh = plsc.VectorSubcoreMesh(core_axis_name="c", subcore_axis_name="s")

    @functools.partial(
        pl.kernel,
        mesh=mesh,
        out_type=jax.ShapeDtypeStruct((_N,), jnp.float32),
        scratch_types=[
            pltpu.VMEM((_CHUNK,), jnp.float32),
            pltpu.VMEM((_CHUNK,), jnp.int32),
            pltpu.VMEM((_L,), jnp.float32),
            pltpu.VMEM((_CHUNK,), jnp.float32),
            pltpu.SemaphoreType.DMA,
        ],
    )
    def sc_kernel(scores_hbm, mask_hbm, bias_hbm, out_hbm,
                  scores_v, mask_v, bias_v, out_v, sem):
        wid = lax.axis_index("s") * _NC + lax.axis_index("c")
        base = wid * _CHUNK
        d1 = pltpu.async_copy(scores_hbm.at[pl.ds(base, _CHUNK)], scores_v, sem)
        d2 = pltpu.async_copy(mask_hbm.at[pl.ds(base, _CHUNK)], mask_v, sem)
        d3 = pltpu.async_copy(bias_hbm, bias_v, sem)
        d1.wait()
        d2.wait()
        d3.wait()
        bias_vec = bias_v[...]
        for i in range(_CHUNK // _L):
            sl = pl.ds(i * _L, _L)
            s = scores_v[sl]
            m = mask_v[sl].astype(jnp.float32)
            out_v[sl] = (s + bias_vec) * m
        pltpu.sync_copy(out_v, out_hbm.at[pl.ds(base, _CHUNK)])

    return sc_kernel


_SC_CALL = _make_sc_call()


@jax.jit
def kernel(sent_group_scores, sel_sent_emb, sel_sent_masks, group_embs,
           candi_sent_masks, bias):
    scores = sent_group_scores.reshape(_N)
    mask = candi_sent_masks.reshape(_N)
    bias_vec = jnp.broadcast_to(jnp.reshape(bias, (1,)), (_L,))
    out = _SC_CALL(scores, mask, bias_vec)
    return out.reshape(_B, _S)


# speedup vs baseline: 1.1129x; 1.0737x over previous
import functools

import jax
import jax.numpy as jnp
from jax import lax
from jax.experimental import pallas as pl
from jax.experimental.pallas import tpu as pltpu, tpu_sc as plsc

_INFO = plsc.get_sparse_core_info()
_NC, _NS, _L = _INFO.num_cores, _INFO.num_subcores, _INFO.num_lanes
_NW = 1 * _NS

_B, _S = 16, 2048
_N = _B * _S
_CHUNK = _N // _NW


def _make_sc_call():
    mesh = plsc.VectorSubcoreMesh(core_axis_name="c", subcore_axis_name="s",
                                  num_cores=1)

    @functools.partial(
        pl.kernel,
        mesh=mesh,
        out_type=jax.ShapeDtypeStruct((_N,), jnp.float32),
        scratch_types=[
            pltpu.VMEM((_CHUNK,), jnp.float32),
            pltpu.VMEM((_CHUNK,), jnp.int32),
            pltpu.VMEM((_L,), jnp.float32),
            pltpu.VMEM((_CHUNK,), jnp.float32),
            pltpu.SemaphoreType.DMA,
        ],
    )
    def sc_kernel(scores_hbm, mask_hbm, bias_hbm, out_hbm,
                  scores_v, mask_v, bias_v, out_v, sem):
        wid = lax.axis_index("s") + lax.axis_index("c")
        base = wid * _CHUNK
        d1 = pltpu.async_copy(scores_hbm.at[pl.ds(base, _CHUNK)], scores_v, sem)
        d2 = pltpu.async_copy(mask_hbm.at[pl.ds(base, _CHUNK)], mask_v, sem)
        d3 = pltpu.async_copy(bias_hbm, bias_v, sem)
        d1.wait()
        d2.wait()
        d3.wait()
        bias_vec = bias_v[...]
        for i in range(_CHUNK // _L):
            sl = pl.ds(i * _L, _L)
            s = scores_v[sl]
            m = mask_v[sl].astype(jnp.float32)
            out_v[sl] = (s + bias_vec) * m
        pltpu.sync_copy(out_v, out_hbm.at[pl.ds(base, _CHUNK)])

    return sc_kernel


_SC_CALL = _make_sc_call()


@jax.jit
def kernel(sent_group_scores, sel_sent_emb, sel_sent_masks, group_embs,
           candi_sent_masks, bias):
    scores = sent_group_scores.reshape(_N)
    mask = candi_sent_masks.reshape(_N)
    bias_vec = jnp.broadcast_to(jnp.reshape(bias, (1,)), (_L,))
    out = _SC_CALL(scores, mask, bias_vec)
    return out.reshape(_B, _S)


# speedup vs baseline: 1.2109x; 1.0881x over previous
import functools

import jax
import jax.numpy as jnp
from jax import lax
from jax.experimental import pallas as pl
from jax.experimental.pallas import tpu as pltpu, tpu_sc as plsc

_INFO = plsc.get_sparse_core_info()
_NS, _L = _INFO.num_subcores, _INFO.num_lanes

_B, _S = 16, 2048


def _make_sc_call():
    mesh = plsc.VectorSubcoreMesh(core_axis_name="c", subcore_axis_name="s",
                                  num_cores=1)

    @functools.partial(
        pl.kernel,
        mesh=mesh,
        out_type=jax.ShapeDtypeStruct((_B, _S), jnp.float32),
        scratch_types=[
            pltpu.VMEM((_S,), jnp.float32),
            pltpu.VMEM((_S,), jnp.int32),
            pltpu.VMEM((_L,), jnp.float32),
            pltpu.VMEM((_S,), jnp.float32),
            pltpu.SemaphoreType.DMA,
        ],
    )
    def sc_kernel(scores_hbm, mask_hbm, bias_hbm, out_hbm,
                  scores_v, mask_v, bias_v, out_v, sem):
        row = lax.axis_index("s") + lax.axis_index("c")
        d1 = pltpu.async_copy(scores_hbm.at[row], scores_v, sem)
        d2 = pltpu.async_copy(mask_hbm.at[row], mask_v, sem)
        d3 = pltpu.async_copy(bias_hbm, bias_v, sem)
        d1.wait()
        d2.wait()
        d3.wait()
        bias_vec = bias_v[...]
        for i in range(_S // _L):
            sl = pl.ds(i * _L, _L)
            out_v[sl] = (scores_v[sl] + bias_vec) * mask_v[sl].astype(jnp.float32)
        pltpu.sync_copy(out_v, out_hbm.at[row])

    return sc_kernel


_SC_CALL = _make_sc_call()


@jax.jit
def kernel(sent_group_scores, sel_sent_emb, sel_sent_masks, group_embs,
           candi_sent_masks, bias):
    bias_vec = jnp.broadcast_to(jnp.reshape(bias, (1,)), (_L,))
    return _SC_CALL(sent_group_scores, candi_sent_masks, bias_vec)


# speedup vs baseline: 1.2545x; 1.0360x over previous
import functools

import jax
import jax.numpy as jnp
from jax import lax
from jax.experimental import pallas as pl
from jax.experimental.pallas import tpu as pltpu, tpu_sc as plsc

_INFO = plsc.get_sparse_core_info()
_NS, _L = _INFO.num_subcores, _INFO.num_lanes

_B, _S = 16, 2048
_HALF = _S // 2


def _make_sc_call():
    mesh = plsc.VectorSubcoreMesh(core_axis_name="c", subcore_axis_name="s",
                                  num_cores=1)

    @functools.partial(
        pl.kernel,
        mesh=mesh,
        out_type=jax.ShapeDtypeStruct((_B, _S), jnp.float32),
        scratch_types=[
            pltpu.VMEM((_S,), jnp.float32),
            pltpu.VMEM((_S,), jnp.int32),
            pltpu.VMEM((_L,), jnp.float32),
            pltpu.VMEM((_S,), jnp.float32),
            pltpu.SemaphoreType.DMA,
            pltpu.SemaphoreType.DMA,
        ],
    )
    def sc_kernel(scores_hbm, mask_hbm, bias_hbm, out_hbm,
                  scores_v, mask_v, bias_v, out_v, sem, out_sem):
        row = lax.axis_index("s") + lax.axis_index("c")
        d1 = pltpu.async_copy(scores_hbm.at[row], scores_v, sem)
        d2 = pltpu.async_copy(mask_hbm.at[row], mask_v, sem)
        d3 = pltpu.async_copy(bias_hbm, bias_v.at[pl.ds(0, 1)], sem)
        d1.wait()
        d2.wait()
        d3.wait()
        bias_vec = jnp.full((_L,), bias_v[...][0], dtype=jnp.float32)
        for i in range(_HALF // _L):
            sl = pl.ds(i * _L, _L)
            out_v[sl] = (scores_v[sl] + bias_vec) * mask_v[sl].astype(jnp.float32)
        w1 = pltpu.async_copy(out_v.at[pl.ds(0, _HALF)],
                              out_hbm.at[row, pl.ds(0, _HALF)], out_sem)
        for i in range(_HALF // _L, _S // _L):
            sl = pl.ds(i * _L, _L)
            out_v[sl] = (scores_v[sl] + bias_vec) * mask_v[sl].astype(jnp.float32)
        w2 = pltpu.async_copy(out_v.at[pl.ds(_HALF, _HALF)],
                              out_hbm.at[row, pl.ds(_HALF, _HALF)], out_sem)
        w1.wait()
        w2.wait()

    return sc_kernel


_SC_CALL = _make_sc_call()


@jax.jit
def kernel(sent_group_scores, sel_sent_emb, sel_sent_masks, group_embs,
           candi_sent_masks, bias):
    return _SC_CALL(sent_group_scores, candi_sent_masks,
                    jnp.reshape(bias, (1,)))


# speedup vs baseline: 1.2643x; 1.0078x over previous
import functools

import jax
import jax.numpy as jnp
from jax import lax
from jax.experimental import pallas as pl
from jax.experimental.pallas import tpu as pltpu, tpu_sc as plsc

_INFO = plsc.get_sparse_core_info()
_NS, _L = _INFO.num_subcores, _INFO.num_lanes

_B, _S = 16, 2048
_HALF = _S // 2


def _make_sc_call():
    mesh = plsc.VectorSubcoreMesh(core_axis_name="c", subcore_axis_name="s",
                                  num_cores=1)

    @functools.partial(
        pl.kernel,
        mesh=mesh,
        out_type=jax.ShapeDtypeStruct((_B, _S), jnp.float32),
        scratch_types=[
            pltpu.VMEM((_S,), jnp.float32),
            pltpu.VMEM((_S,), jnp.int32),
            pltpu.VMEM((_L,), jnp.float32),
            pltpu.VMEM((_S,), jnp.float32),
            pltpu.SemaphoreType.DMA,
            pltpu.SemaphoreType.DMA,
        ],
    )
    def sc_kernel(scores_hbm, mask_hbm, bias_hbm, out_hbm,
                  scores_v, mask_v, bias_v, out_v, sem, out_sem):
        row = lax.axis_index("s") + lax.axis_index("c")
        h0 = pl.ds(0, _HALF)
        h1 = pl.ds(_HALF, _HALF)
        d3 = pltpu.async_copy(bias_hbm, bias_v.at[pl.ds(0, 1)], sem)
        s0 = pltpu.async_copy(scores_hbm.at[row, h0], scores_v.at[h0], sem)
        m0 = pltpu.async_copy(mask_hbm.at[row, h0], mask_v.at[h0], sem)
        s1 = pltpu.async_copy(scores_hbm.at[row, h1], scores_v.at[h1], sem)
        m1 = pltpu.async_copy(mask_hbm.at[row, h1], mask_v.at[h1], sem)
        d3.wait()
        s0.wait()
        m0.wait()
        bias_vec = jnp.full((_L,), bias_v[...][0], dtype=jnp.float32)
        for i in range(_HALF // _L):
            sl = pl.ds(i * _L, _L)
            out_v[sl] = (scores_v[sl] + bias_vec) * mask_v[sl].astype(jnp.float32)
        w1 = pltpu.async_copy(out_v.at[h0], out_hbm.at[row, h0], out_sem)
        s1.wait()
        m1.wait()
        for i in range(_HALF // _L, _S // _L):
            sl = pl.ds(i * _L, _L)
            out_v[sl] = (scores_v[sl] + bias_vec) * mask_v[sl].astype(jnp.float32)
        w2 = pltpu.async_copy(out_v.at[h1], out_hbm.at[row, h1], out_sem)
        w1.wait()
        w2.wait()

    return sc_kernel


_SC_CALL = _make_sc_call()


@jax.jit
def kernel(sent_group_scores, sel_sent_emb, sel_sent_masks, group_embs,
           candi_sent_masks, bias):
    return _SC_CALL(sent_group_scores, candi_sent_masks,
                    jnp.reshape(bias, (1,)))
